# baseline (speedup 1.0000x reference)
import jax
import jax.numpy as jnp
from jax.experimental import pallas as pl
from jax.experimental.pallas import tpu as pltpu

T = 2048
E = 8
D = 1024
H = 512

BT = 256


def _gbutterfly(x, lane_e, op):
    n = x.shape[1]
    for d in (4, 2, 1):
        sel = (lane_e & d) == 0
        partner = jnp.where(sel, pltpu.roll(x, n - d, 1), pltpu.roll(x, d, 1))
        x = op(x, partner)
    return x


def _gate_kernel(x_ref, w1_ref, b1_ref, w2_ref, b2_ref, w3_ref, scal_ref,
                 soft_ref, hard_ref):
    h = jnp.dot(x_ref[...], w1_ref[...], preferred_element_type=jnp.float32)
    h = jnp.maximum(h + b1_ref[...], 0.0)
    h = jnp.dot(h, w2_ref[...], preferred_element_type=jnp.float32)
    h = jnp.maximum(h + b2_ref[...], 0.0)
    logit = jax.lax.dot_general(
        w3_ref[...], h, (((0,), (1,)), ((), ())),
        preferred_element_type=jnp.float32)
    b3 = scal_ref[0, 0]
    inv_t = scal_ref[0, 1]
    y = (logit + b3) * inv_t
    R = BT * E
    lane_e = jax.lax.broadcasted_iota(jnp.int32, (1, R), 1) & (E - 1)
    m = _gbutterfly(y, lane_e, jnp.maximum)
    e = jnp.exp(y - m)
    s = _gbutterfly(e, lane_e, jnp.add)
    soft = e / s
    soft_ref[...] = soft
    ms = _gbutterfly(soft, lane_e, jnp.maximum)
    win = _gbutterfly(jnp.where(soft == ms, lane_e, E), lane_e, jnp.minimum)
    hard_ref[...] = jnp.where(lane_e == win, 1.0, 0.0).astype(jnp.float32)


def kernel(x_z, W1, b1, W2, b2, W3, b3, temperature):
    x2d = x_z.reshape(T * E, D)
    b1r = b1.reshape(1, H)
    b2r = b2.reshape(1, H)
    w3r = W3.reshape(H, 1)
    scal = jnp.stack([b3[0], 1.0 / temperature]).reshape(1, 2).astype(jnp.float32)

    R = BT * E
    soft, hard = pl.pallas_call(
        _gate_kernel,
        grid=(T // BT,),
        in_specs=[
            pl.BlockSpec((R, D), lambda i: (i, 0)),
            pl.BlockSpec((D, H), lambda i: (0, 0)),
            pl.BlockSpec((1, H), lambda i: (0, 0)),
            pl.BlockSpec((H, H), lambda i: (0, 0)),
            pl.BlockSpec((1, H), lambda i: (0, 0)),
            pl.BlockSpec((H, 1), lambda i: (0, 0)),
            pl.BlockSpec((1, 2), lambda i: (0, 0)),
        ],
        out_specs=[
            pl.BlockSpec((1, R), lambda i: (0, i)),
            pl.BlockSpec((1, R), lambda i: (0, i)),
        ],
        out_shape=[
            jax.ShapeDtypeStruct((1, T * E), jnp.float32),
            jax.ShapeDtypeStruct((1, T * E), jnp.float32),
        ],
    )(x2d, W1, b1r, W2, b2r, w3r, scal)
    return soft.reshape(T, E, 1), hard.reshape(T, E, 1)


# speedup vs baseline: 1.0853x; 1.0853x over previous
import jax
import jax.numpy as jnp
from jax.experimental import pallas as pl
from jax.experimental.pallas import tpu as pltpu

T = 2048
E = 8
D = 1024
H = 512

BT = 256


def _gbutterfly(x, lane_e, op):
    n = x.shape[1]
    for d in (4, 2, 1):
        sel = (lane_e & d) == 0
        partner = jnp.where(sel, pltpu.roll(x, n - d, 1), pltpu.roll(x, d, 1))
        x = op(x, partner)
    return x


def _gate_kernel(x_ref, w1_ref, b1_ref, w2_ref, b2_ref, w3_ref, scal_ref,
                 soft_ref, hard_ref, logit_ref):
    i = pl.program_id(0)
    nsteps = pl.num_programs(0)
    R = BT * E
    h = jnp.dot(x_ref[...], w1_ref[...], preferred_element_type=jnp.float32)
    h = jnp.maximum(h + b1_ref[...], 0.0)
    h = jnp.dot(h, w2_ref[...], preferred_element_type=jnp.float32)
    h = jnp.maximum(h + b2_ref[...], 0.0)
    logit = jax.lax.dot_general(
        w3_ref[...], h, (((0,), (1,)), ((), ())),
        preferred_element_type=jnp.float32)
    logit_ref[:, pl.ds(i * R, R)] = logit

    @pl.when(i == nsteps - 1)
    def _epilogue():
        b3 = scal_ref[0, 0]
        inv_t = scal_ref[0, 1]
        y = (logit_ref[...] + b3) * inv_t
        lane_e = jax.lax.broadcasted_iota(jnp.int32, y.shape, 1) & (E - 1)
        m = _gbutterfly(y, lane_e, jnp.maximum)
        e = jnp.exp(y - m)
        s = _gbutterfly(e, lane_e, jnp.add)
        soft = e / s
        soft_ref[...] = soft
        ms = _gbutterfly(soft, lane_e, jnp.maximum)
        win = _gbutterfly(jnp.where(soft == ms, lane_e, E), lane_e,
                          jnp.minimum)
        hard_ref[...] = jnp.where(lane_e == win, 1.0, 0.0).astype(jnp.float32)


def kernel(x_z, W1, b1, W2, b2, W3, b3, temperature):
    x2d = x_z.reshape(T * E, D)
    b1r = b1.reshape(1, H)
    b2r = b2.reshape(1, H)
    scal = jnp.stack([b3[0], 1.0 / temperature]).reshape(1, 2).astype(jnp.float32)

    R = BT * E
    soft, hard = pl.pallas_call(
        _gate_kernel,
        grid=(T // BT,),
        in_specs=[
            pl.BlockSpec((R, D), lambda i: (i, 0)),
            pl.BlockSpec((D, H), lambda i: (0, 0)),
            pl.BlockSpec((1, H), lambda i: (0, 0)),
            pl.BlockSpec((H, H), lambda i: (0, 0)),
            pl.BlockSpec((1, H), lambda i: (0, 0)),
            pl.BlockSpec((H, 1), lambda i: (0, 0)),
            pl.BlockSpec((1, 2), lambda i: (0, 0)),
        ],
        out_specs=[
            pl.BlockSpec((1, T * E), lambda i: (0, 0)),
            pl.BlockSpec((1, T * E), lambda i: (0, 0)),
        ],
        out_shape=[
            jax.ShapeDtypeStruct((1, T * E), jnp.float32),
            jax.ShapeDtypeStruct((1, T * E), jnp.float32),
        ],
        scratch_shapes=[pltpu.VMEM((1, T * E), jnp.float32)],
    )(x2d, W1, b1r, W2, b2r, W3.reshape(H, 1), scal)
    return soft.reshape(T, E, 1), hard.reshape(T, E, 1)
